# Initial kernel scaffold; baseline (speedup 1.0000x reference)
import jax
import jax.numpy as jnp
from jax.experimental import pallas as pl


def kernel(x, table):
    raise NotImplementedError("write your pallas kernel here")


# speedup vs baseline: 8.1032x; 8.1032x over previous
import functools

import jax
import jax.numpy as jnp
from jax import lax
from jax.experimental import pallas as pl
from jax.experimental.pallas import tpu as pltpu
from jax.experimental.pallas import tpu_sc as plsc


def _build_encoder(B, L, D, table_rows):
    info = plsc.get_sparse_core_info()
    NC, NS, LANES = info.num_cores, info.num_subcores, info.num_lanes
    NW = NC * NS
    assert B % NW == 0
    b_per_w = B // NW
    CB = 16
    assert b_per_w % CB == 0
    nchunks = b_per_w // CB
    CI = CB * L
    assert CI % 8 == 0
    LP = (L + LANES - 1) // LANES * LANES
    NWV = LP // LANES
    CW = CB * LP
    ND = D // LANES
    assert D == ND * LANES
    subs = []
    off = 0
    while off < CI:
        sz = min(128, CI - off)
        subs.append((off, sz))
        off += sz

    mesh = plsc.VectorSubcoreMesh(core_axis_name="c", subcore_axis_name="s")

    @functools.partial(
        pl.kernel,
        mesh=mesh,
        out_type=jax.ShapeDtypeStruct((B, D), jnp.float32),
        compiler_params=pltpu.CompilerParams(use_tc_tiling_on_sc=False),
        scratch_types=[
            pltpu.VMEM((CI,), jnp.int32),
            pltpu.VMEM((CW,), jnp.float32),
            pltpu.VMEM((CI, D), jnp.float32),
            pltpu.VMEM((CB, D), jnp.float32),
            pltpu.SemaphoreType.DMA,
        ],
    )
    def encode(table_h, idx_h, w_h, out_h, idx_v, w_v, rows_v, out_v, sem):
        wid = lax.axis_index("s") * NC + lax.axis_index("c")
        ibase = wid * b_per_w * L
        wbase = wid * b_per_w * LP

        def chunk(c, carry):
            pltpu.sync_copy(idx_h.at[pl.ds(ibase + c * CI, CI)], idx_v)
            pltpu.sync_copy(w_h.at[pl.ds(wbase + c * CW, CW)], w_v)
            copies = [
                pltpu.async_copy(
                    table_h.at[idx_v.at[pl.ds(o, s)]],
                    rows_v.at[pl.ds(o, s)],
                    sem,
                )
                for o, s in subs
            ]
            for cp in copies:
                cp.wait()

            def row(i, carry2):
                accs = [jnp.zeros((LANES,), jnp.float32) for _ in range(ND)]
                wvecs = [
                    w_v[pl.ds(i * LP + k * LANES, LANES)] for k in range(NWV)
                ]
                for l in range(L):
                    r = i * L + l
                    wv = wvecs[l // LANES][l % LANES]
                    for d in range(ND):
                        accs[d] = accs[d] + wv * rows_v[r, pl.ds(d * LANES, LANES)]
                for d in range(ND):
                    out_v[i, pl.ds(d * LANES, LANES)] = accs[d]
                return carry2

            lax.fori_loop(0, CB, row, 0)
            pltpu.sync_copy(out_v, out_h.at[pl.ds(wid * b_per_w + c * CB, CB)])
            return carry

        lax.fori_loop(0, nchunks, chunk, 0)

    return encode


def kernel(x, table):
    B, two_l = x.shape
    L = two_l // 2
    V, D = table.shape
    idx = x[:, :L].astype(jnp.int32).reshape(-1)
    LP = (L + 15) // 16 * 16
    w = jnp.pad(x[:, L:], ((0, 0), (0, LP - L))).reshape(-1)
    enc = _build_encoder(B, L, D, V)
    return enc(table, idx, w)


# speedup vs baseline: 9.5063x; 1.1732x over previous
import functools

import jax
import jax.numpy as jnp
from jax import lax
from jax.experimental import pallas as pl
from jax.experimental.pallas import tpu as pltpu
from jax.experimental.pallas import tpu_sc as plsc


def _build_encoder(B, L, D, table_rows):
    info = plsc.get_sparse_core_info()
    NC, NS, LANES = info.num_cores, info.num_subcores, info.num_lanes
    NW = NC * NS
    assert B % NW == 0
    b_per_w = B // NW
    CB = 16
    assert b_per_w % CB == 0
    nchunks = b_per_w // CB
    CI = CB * L
    assert CI % 8 == 0
    LP = (L + LANES - 1) // LANES * LANES
    NWV = LP // LANES
    CW = CB * LP
    ND = D // LANES
    assert D == ND * LANES
    subs = []
    off = 0
    while off < CI:
        sz = min(128, CI - off)
        subs.append((off, sz))
        off += sz

    mesh = plsc.VectorSubcoreMesh(core_axis_name="c", subcore_axis_name="s")

    @functools.partial(
        pl.kernel,
        mesh=mesh,
        out_type=jax.ShapeDtypeStruct((B, D), jnp.float32),
        compiler_params=pltpu.CompilerParams(use_tc_tiling_on_sc=False),
        scratch_types=[
            pltpu.VMEM((b_per_w * L,), jnp.int32),
            pltpu.VMEM((b_per_w * LP,), jnp.float32),
            pltpu.VMEM((CI, D), jnp.float32),
            pltpu.VMEM((CI, D), jnp.float32),
            pltpu.VMEM((CB, D), jnp.float32),
            pltpu.SemaphoreType.DMA,
            pltpu.SemaphoreType.DMA,
        ],
    )
    def encode(table_h, idx_h, w_h, out_h, idx_v, w_v, rows0, rows1, out_v,
               sem0, sem1):
        wid = lax.axis_index("s") * NC + lax.axis_index("c")
        bufs = (rows0, rows1)
        sems = (sem0, sem1)

        pltpu.sync_copy(idx_h.at[pl.ds(wid * b_per_w * L, b_per_w * L)], idx_v)
        pltpu.sync_copy(w_h.at[pl.ds(wid * b_per_w * LP, b_per_w * LP)], w_v)

        def fire(c):
            buf, sem = bufs[c % 2], sems[c % 2]
            for o, s in subs:
                pltpu.async_copy(
                    table_h.at[idx_v.at[pl.ds(c * CI + o, s)]],
                    buf.at[pl.ds(o, s)],
                    sem,
                )

        def drain(c):
            buf, sem = bufs[c % 2], sems[c % 2]
            for o, s in subs:
                pltpu.make_async_copy(
                    table_h.at[idx_v.at[pl.ds(c * CI + o, s)]],
                    buf.at[pl.ds(o, s)],
                    sem,
                ).wait()

        fire(0)
        for c in range(nchunks):
            if c + 1 < nchunks:
                fire(c + 1)
            drain(c)
            rows_v = bufs[c % 2]

            def row(i, carry, c=c, rows_v=rows_v):
                accs = [jnp.zeros((LANES,), jnp.float32) for _ in range(ND)]
                wvecs = [
                    w_v[pl.ds(c * CW + i * LP + k * LANES, LANES)]
                    for k in range(NWV)
                ]
                for l in range(L):
                    r = i * L + l
                    wl = wvecs[l // LANES][l % LANES]
                    for d in range(ND):
                        accs[d] = accs[d] + wl * rows_v[r, pl.ds(d * LANES, LANES)]
                for d in range(ND):
                    out_v[i, pl.ds(d * LANES, LANES)] = accs[d]
                return carry

            lax.fori_loop(0, CB, row, 0)
            pltpu.sync_copy(out_v, out_h.at[pl.ds(wid * b_per_w + c * CB, CB)])

    return encode


def kernel(x, table):
    B, two_l = x.shape
    L = two_l // 2
    V, D = table.shape
    idx = x[:, :L].astype(jnp.int32).reshape(-1)
    LP = (L + 15) // 16 * 16
    w = jnp.pad(x[:, L:], ((0, 0), (0, LP - L))).reshape(-1)
    enc = _build_encoder(B, L, D, V)
    return enc(table, idx, w)
